# baseline (speedup 1.0000x reference)
import math

import jax
import jax.numpy as jnp
from jax.experimental import pallas as pl

L = 64
K = 128
B = 4096
P = K // 2
_HALF_LOG_2PI = 0.5 * math.log(2.0 * math.pi)
_LOG2E = 1.4426950408889634
_LN2 = 0.6931471805599453


def _mog_block(z_ref, mP_ref, lvP_ref, wP_ref, wrow_ref, out_ref):
    Bb = z_ref.shape[0]
    zb = z_ref[...]
    zd = jnp.concatenate([zb, zb], axis=1)
    zd2 = zd * zd

    wrow = wrow_ref[...]
    wm = jnp.max(wrow)
    z_norm = wm + jnp.log(jnp.sum(jnp.exp(wrow - wm)))

    lvP = lvP_ref[...]
    mP = mP_ref[...]
    ha = 0.5 * jnp.exp(-lvP)
    c = (wP_ref[...] - z_norm - _HALF_LOG_2PI) - 0.5 * lvP

    cmax = jnp.max(c, axis=0, keepdims=True)
    cmax = jnp.maximum(cmax, jnp.concatenate(
        [cmax[:, L:], cmax[:, :L]], axis=1))
    G2 = _LOG2E * cmax

    q2 = (-_LOG2E) * ha
    q1 = (2.0 * _LOG2E) * (ha * mP)
    q0 = _LOG2E * (c - ha * (mP * mP)) - G2

    s = jnp.zeros((Bb, 2 * L), dtype=jnp.float32)
    for p in range(P):
        y = (q0[p:p + 1, :] + zd * q1[p:p + 1, :]) + zd2 * q2[p:p + 1, :]
        s = s + jnp.exp2(y)

    st = s[:, :L] + s[:, L:]
    out_ref[...] = _LN2 * (G2[:, :L] + jnp.log2(st))


@jax.jit
def kernel(z, means, logvars, w):
    mP = means.reshape(P, 2 * L)
    lvP = logvars.reshape(P, 2 * L)
    wflat = w.reshape(K)
    wPp = jnp.broadcast_to(wflat.reshape(P, 2, 1), (P, 2, L)).reshape(P, 2 * L)
    wrow = wflat.reshape(1, K)

    Bb = 64
    grid = (B // Bb,)
    out = pl.pallas_call(
        _mog_block,
        grid=grid,
        in_specs=[
            pl.BlockSpec((Bb, L), lambda i: (i, 0)),
            pl.BlockSpec((P, 2 * L), lambda i: (0, 0)),
            pl.BlockSpec((P, 2 * L), lambda i: (0, 0)),
            pl.BlockSpec((P, 2 * L), lambda i: (0, 0)),
            pl.BlockSpec((1, K), lambda i: (0, 0)),
        ],
        out_specs=pl.BlockSpec((Bb, L), lambda i: (i, 0)),
        out_shape=jax.ShapeDtypeStruct((B, L), jnp.float32),
    )(z, mP, lvP, wPp, wrow)
    return out.T


# speedup vs baseline: 1.2725x; 1.2725x over previous
import math

import jax
import jax.numpy as jnp
from jax.experimental import pallas as pl

L = 64
K = 128
B = 4096
P = K // 2
_HALF_LOG_2PI = 0.5 * math.log(2.0 * math.pi)
_LOG2E = 1.4426950408889634
_LN2 = 0.6931471805599453


def _mog_block(z_ref, mP_ref, lvP_ref, wP_ref, wrow_ref, out_ref):
    Bb = z_ref.shape[0]
    zb = z_ref[...]
    zd = jnp.concatenate([zb, zb], axis=1)
    zd2 = zd * zd

    wrow = wrow_ref[...]
    wm = jnp.max(wrow)
    z_norm = wm + jnp.log(jnp.sum(jnp.exp(wrow - wm)))

    lvP = lvP_ref[...]
    mP = mP_ref[...]
    ha = 0.5 * jnp.exp(-lvP)
    c = (wP_ref[...] - z_norm - _HALF_LOG_2PI) - 0.5 * lvP

    cmax = jnp.max(c, axis=0, keepdims=True)
    cmax = jnp.maximum(cmax, jnp.concatenate(
        [cmax[:, L:], cmax[:, :L]], axis=1))
    G2 = _LOG2E * cmax

    q2 = (-_LOG2E) * ha
    q1 = (2.0 * _LOG2E) * (ha * mP)
    q0 = _LOG2E * (c - ha * (mP * mP)) - G2

    s = jnp.zeros((Bb, 2 * L), dtype=jnp.float32)
    for p in range(P):
        y = (q0[p:p + 1, :] + zd * q1[p:p + 1, :]) + zd2 * q2[p:p + 1, :]
        s = s + jnp.exp2(y)

    st = s[:, :L] + s[:, L:]
    out_ref[...] = _LN2 * (G2[:, :L] + jnp.log2(st))


@jax.jit
def kernel(z, means, logvars, w):
    mP = means.reshape(P, 2 * L)
    lvP = logvars.reshape(P, 2 * L)
    wflat = w.reshape(K)
    wPp = jnp.broadcast_to(wflat.reshape(P, 2, 1), (P, 2, L)).reshape(P, 2 * L)
    wrow = wflat.reshape(1, K)

    Bb = 128
    grid = (B // Bb,)
    out = pl.pallas_call(
        _mog_block,
        grid=grid,
        in_specs=[
            pl.BlockSpec((Bb, L), lambda i: (i, 0)),
            pl.BlockSpec((P, 2 * L), lambda i: (0, 0)),
            pl.BlockSpec((P, 2 * L), lambda i: (0, 0)),
            pl.BlockSpec((P, 2 * L), lambda i: (0, 0)),
            pl.BlockSpec((1, K), lambda i: (0, 0)),
        ],
        out_specs=pl.BlockSpec((Bb, L), lambda i: (i, 0)),
        out_shape=jax.ShapeDtypeStruct((B, L), jnp.float32),
    )(z, mP, lvP, wPp, wrow)
    return out.T


# speedup vs baseline: 1.4766x; 1.1604x over previous
import math

import jax
import jax.numpy as jnp
from jax.experimental import pallas as pl

L = 64
K = 128
B = 4096
P = K // 2
_HALF_LOG_2PI = 0.5 * math.log(2.0 * math.pi)
_LOG2E = 1.4426950408889634
_LN2 = 0.6931471805599453


def _mog_block(z_ref, mP_ref, lvP_ref, wP_ref, wrow_ref, out_ref):
    Bb = z_ref.shape[0]
    zb = z_ref[...]
    zd = jnp.concatenate([zb, zb], axis=1)
    zd2 = zd * zd

    wrow = wrow_ref[...]
    wm = jnp.max(wrow)
    z_norm = wm + jnp.log(jnp.sum(jnp.exp(wrow - wm)))

    lvP = lvP_ref[...]
    mP = mP_ref[...]
    ha = 0.5 * jnp.exp(-lvP)
    c = (wP_ref[...] - z_norm - _HALF_LOG_2PI) - 0.5 * lvP

    cmax = jnp.max(c, axis=0, keepdims=True)
    cmax = jnp.maximum(cmax, jnp.concatenate(
        [cmax[:, L:], cmax[:, :L]], axis=1))
    G2 = _LOG2E * cmax

    q2 = (-_LOG2E) * ha
    q1 = (2.0 * _LOG2E) * (ha * mP)
    q0 = _LOG2E * (c - ha * (mP * mP)) - G2

    s = jnp.zeros((Bb, 2 * L), dtype=jnp.float32)
    for p in range(P):
        y = (q0[p:p + 1, :] + zd * q1[p:p + 1, :]) + zd2 * q2[p:p + 1, :]
        s = s + jnp.exp2(y)

    st = s[:, :L] + s[:, L:]
    out_ref[...] = _LN2 * (G2[:, :L] + jnp.log2(st))


@jax.jit
def kernel(z, means, logvars, w):
    mP = means.reshape(P, 2 * L)
    lvP = logvars.reshape(P, 2 * L)
    wflat = w.reshape(K)
    wPp = jnp.broadcast_to(wflat.reshape(P, 2, 1), (P, 2, L)).reshape(P, 2 * L)
    wrow = wflat.reshape(1, K)

    Bb = 256
    grid = (B // Bb,)
    out = pl.pallas_call(
        _mog_block,
        grid=grid,
        in_specs=[
            pl.BlockSpec((Bb, L), lambda i: (i, 0)),
            pl.BlockSpec((P, 2 * L), lambda i: (0, 0)),
            pl.BlockSpec((P, 2 * L), lambda i: (0, 0)),
            pl.BlockSpec((P, 2 * L), lambda i: (0, 0)),
            pl.BlockSpec((1, K), lambda i: (0, 0)),
        ],
        out_specs=pl.BlockSpec((Bb, L), lambda i: (i, 0)),
        out_shape=jax.ShapeDtypeStruct((B, L), jnp.float32),
    )(z, mP, lvP, wPp, wrow)
    return out.T


# speedup vs baseline: 1.6040x; 1.0862x over previous
import math

import jax
import jax.numpy as jnp
from jax.experimental import pallas as pl

L = 64
K = 128
B = 4096
P = K // 2
_HALF_LOG_2PI = 0.5 * math.log(2.0 * math.pi)
_LOG2E = 1.4426950408889634
_LN2 = 0.6931471805599453


def _mog_block(z_ref, mP_ref, lvP_ref, wP_ref, wrow_ref, out_ref):
    Bb = z_ref.shape[0]
    zb = z_ref[...]
    zd = jnp.concatenate([zb, zb], axis=1)
    zd2 = zd * zd

    wrow = wrow_ref[...]
    wm = jnp.max(wrow)
    z_norm = wm + jnp.log(jnp.sum(jnp.exp(wrow - wm)))

    lvP = lvP_ref[...]
    mP = mP_ref[...]
    ha = 0.5 * jnp.exp(-lvP)
    c = (wP_ref[...] - z_norm - _HALF_LOG_2PI) - 0.5 * lvP

    cmax = jnp.max(c, axis=0, keepdims=True)
    cmax = jnp.maximum(cmax, jnp.concatenate(
        [cmax[:, L:], cmax[:, :L]], axis=1))
    G2 = _LOG2E * cmax

    q2 = (-_LOG2E) * ha
    q1 = (2.0 * _LOG2E) * (ha * mP)
    q0 = _LOG2E * (c - ha * (mP * mP)) - G2

    s = jnp.zeros((Bb, 2 * L), dtype=jnp.float32)
    for p in range(P):
        y = (q0[p:p + 1, :] + zd * q1[p:p + 1, :]) + zd2 * q2[p:p + 1, :]
        s = s + jnp.exp2(y)

    st = s[:, :L] + s[:, L:]
    out_ref[...] = _LN2 * (G2[:, :L] + jnp.log2(st))


@jax.jit
def kernel(z, means, logvars, w):
    mP = means.reshape(P, 2 * L)
    lvP = logvars.reshape(P, 2 * L)
    wflat = w.reshape(K)
    wPp = jnp.broadcast_to(wflat.reshape(P, 2, 1), (P, 2, L)).reshape(P, 2 * L)
    wrow = wflat.reshape(1, K)

    Bb = 512
    grid = (B // Bb,)
    out = pl.pallas_call(
        _mog_block,
        grid=grid,
        in_specs=[
            pl.BlockSpec((Bb, L), lambda i: (i, 0)),
            pl.BlockSpec((P, 2 * L), lambda i: (0, 0)),
            pl.BlockSpec((P, 2 * L), lambda i: (0, 0)),
            pl.BlockSpec((P, 2 * L), lambda i: (0, 0)),
            pl.BlockSpec((1, K), lambda i: (0, 0)),
        ],
        out_specs=pl.BlockSpec((Bb, L), lambda i: (i, 0)),
        out_shape=jax.ShapeDtypeStruct((B, L), jnp.float32),
    )(z, mP, lvP, wPp, wrow)
    return out.T


# speedup vs baseline: 1.6588x; 1.0342x over previous
import math

import jax
import jax.numpy as jnp
from jax.experimental import pallas as pl

L = 64
K = 128
B = 4096
P = K // 2
_HALF_LOG_2PI = 0.5 * math.log(2.0 * math.pi)
_LOG2E = 1.4426950408889634
_LN2 = 0.6931471805599453


def _mog_block(z_ref, mP_ref, lvP_ref, wP_ref, wrow_ref, out_ref):
    Bb = z_ref.shape[0]
    zb = z_ref[...]
    zd = jnp.concatenate([zb, zb], axis=1)
    zd2 = zd * zd

    wrow = wrow_ref[...]
    wm = jnp.max(wrow)
    z_norm = wm + jnp.log(jnp.sum(jnp.exp(wrow - wm)))

    lvP = lvP_ref[...]
    mP = mP_ref[...]
    ha = 0.5 * jnp.exp(-lvP)
    c = (wP_ref[...] - z_norm - _HALF_LOG_2PI) - 0.5 * lvP

    cmax = jnp.max(c, axis=0, keepdims=True)
    cmax = jnp.maximum(cmax, jnp.concatenate(
        [cmax[:, L:], cmax[:, :L]], axis=1))
    G2 = _LOG2E * cmax

    q2 = (-_LOG2E) * ha
    q1 = (2.0 * _LOG2E) * (ha * mP)
    q0 = _LOG2E * (c - ha * (mP * mP)) - G2

    s = jnp.zeros((Bb, 2 * L), dtype=jnp.float32)
    for p in range(P):
        y = (q0[p:p + 1, :] + zd * q1[p:p + 1, :]) + zd2 * q2[p:p + 1, :]
        s = s + jnp.exp2(y)

    st = s[:, :L] + s[:, L:]
    out_ref[...] = _LN2 * (G2[:, :L] + jnp.log2(st))


@jax.jit
def kernel(z, means, logvars, w):
    mP = means.reshape(P, 2 * L)
    lvP = logvars.reshape(P, 2 * L)
    wflat = w.reshape(K)
    wPp = jnp.broadcast_to(wflat.reshape(P, 2, 1), (P, 2, L)).reshape(P, 2 * L)
    wrow = wflat.reshape(1, K)

    Bb = 1024
    grid = (B // Bb,)
    out = pl.pallas_call(
        _mog_block,
        grid=grid,
        in_specs=[
            pl.BlockSpec((Bb, L), lambda i: (i, 0)),
            pl.BlockSpec((P, 2 * L), lambda i: (0, 0)),
            pl.BlockSpec((P, 2 * L), lambda i: (0, 0)),
            pl.BlockSpec((P, 2 * L), lambda i: (0, 0)),
            pl.BlockSpec((1, K), lambda i: (0, 0)),
        ],
        out_specs=pl.BlockSpec((Bb, L), lambda i: (i, 0)),
        out_shape=jax.ShapeDtypeStruct((B, L), jnp.float32),
    )(z, mP, lvP, wPp, wrow)
    return out.T


# speedup vs baseline: 1.6787x; 1.0120x over previous
import math

import jax
import jax.numpy as jnp
from jax.experimental import pallas as pl

L = 64
K = 128
B = 4096
P = K // 2
_HALF_LOG_2PI = 0.5 * math.log(2.0 * math.pi)
_LOG2E = 1.4426950408889634
_LN2 = 0.6931471805599453


def _mog_block(z_ref, mP_ref, lvP_ref, wP_ref, wrow_ref, out_ref):
    Bb = z_ref.shape[0]

    wrow = wrow_ref[...]
    wm = jnp.max(wrow)
    z_norm = wm + jnp.log(jnp.sum(jnp.exp(wrow - wm)))

    lvP = lvP_ref[...]
    mP = mP_ref[...]
    ha = 0.5 * jnp.exp(-lvP)
    c = (wP_ref[...] - z_norm - _HALF_LOG_2PI) - 0.5 * lvP

    cmax = jnp.max(c, axis=0, keepdims=True)
    cmax = jnp.maximum(cmax, jnp.concatenate(
        [cmax[:, L:], cmax[:, :L]], axis=1))
    G2 = _LOG2E * cmax

    q2 = (-_LOG2E) * ha
    q1 = (2.0 * _LOG2E) * (ha * mP)
    q0 = _LOG2E * (c - ha * (mP * mP)) - G2

    SUB = 128
    for rc in range(Bb // SUB):
        zb = z_ref[rc * SUB:(rc + 1) * SUB, :]
        zd = jnp.concatenate([zb, zb], axis=1)
        s = jnp.zeros((SUB, 2 * L), dtype=jnp.float32)
        for p in range(P):
            y = q0[p:p + 1, :] + zd * (q1[p:p + 1, :] + zd * q2[p:p + 1, :])
            s = s + jnp.exp2(y)
        st = s[:, :L] + s[:, L:]
        out_ref[rc * SUB:(rc + 1) * SUB, :] = _LN2 * (G2[:, :L] + jnp.log2(st))


@jax.jit
def kernel(z, means, logvars, w):
    mP = means.reshape(P, 2 * L)
    lvP = logvars.reshape(P, 2 * L)
    wflat = w.reshape(K)
    wPp = jnp.broadcast_to(wflat.reshape(P, 2, 1), (P, 2, L)).reshape(P, 2 * L)
    wrow = wflat.reshape(1, K)

    Bb = 1024
    grid = (B // Bb,)
    out = pl.pallas_call(
        _mog_block,
        grid=grid,
        in_specs=[
            pl.BlockSpec((Bb, L), lambda i: (i, 0)),
            pl.BlockSpec((P, 2 * L), lambda i: (0, 0)),
            pl.BlockSpec((P, 2 * L), lambda i: (0, 0)),
            pl.BlockSpec((P, 2 * L), lambda i: (0, 0)),
            pl.BlockSpec((1, K), lambda i: (0, 0)),
        ],
        out_specs=pl.BlockSpec((Bb, L), lambda i: (i, 0)),
        out_shape=jax.ShapeDtypeStruct((B, L), jnp.float32),
    )(z, mP, lvP, wPp, wrow)
    return out.T


# speedup vs baseline: 1.8062x; 1.0760x over previous
import math

import jax
import jax.numpy as jnp
from jax.experimental import pallas as pl

L = 64
K = 128
B = 4096
P = K // 2
_HALF_LOG_2PI = 0.5 * math.log(2.0 * math.pi)
_LOG2E = 1.4426950408889634
_LN2 = 0.6931471805599453


def _mog_block(z_ref, mP_ref, lvP_ref, wP_ref, wrow_ref, out_ref):
    Bb = z_ref.shape[0]

    wrow = wrow_ref[...]
    wm = jnp.max(wrow)
    z_norm = wm + jnp.log(jnp.sum(jnp.exp(wrow - wm)))

    lvP = lvP_ref[...]
    mP = mP_ref[...]
    ha = 0.5 * jnp.exp(-lvP)
    c = (wP_ref[...] - z_norm - _HALF_LOG_2PI) - 0.5 * lvP

    cmax = jnp.max(c, axis=0, keepdims=True)
    cmax = jnp.maximum(cmax, jnp.concatenate(
        [cmax[:, L:], cmax[:, :L]], axis=1))
    G2 = _LOG2E * cmax

    q2 = (-_LOG2E) * ha
    q1 = (2.0 * _LOG2E) * (ha * mP)
    q0 = _LOG2E * (c - ha * (mP * mP)) - G2

    SUB = 128
    for rc in range(Bb // SUB):
        zb = z_ref[rc * SUB:(rc + 1) * SUB, :]
        zd = jnp.concatenate([zb, zb], axis=1)
        s = jnp.zeros((SUB, 2 * L), dtype=jnp.float32)
        for p in range(P):
            y = q0[p:p + 1, :] + zd * (q1[p:p + 1, :] + zd * q2[p:p + 1, :])
            s = s + jnp.exp2(y)
        st = s[:, :L] + s[:, L:]
        res = _LN2 * (G2[:, :L] + jnp.log2(st))
        out_ref[:, rc * SUB:(rc + 1) * SUB] = res.T


@jax.jit
def kernel(z, means, logvars, w):
    mP = means.reshape(P, 2 * L)
    lvP = logvars.reshape(P, 2 * L)
    wflat = w.reshape(K)
    wPp = jnp.broadcast_to(wflat.reshape(P, 2, 1), (P, 2, L)).reshape(P, 2 * L)
    wrow = wflat.reshape(1, K)

    Bb = 1024
    grid = (B // Bb,)
    out = pl.pallas_call(
        _mog_block,
        grid=grid,
        in_specs=[
            pl.BlockSpec((Bb, L), lambda i: (i, 0)),
            pl.BlockSpec((P, 2 * L), lambda i: (0, 0)),
            pl.BlockSpec((P, 2 * L), lambda i: (0, 0)),
            pl.BlockSpec((P, 2 * L), lambda i: (0, 0)),
            pl.BlockSpec((1, K), lambda i: (0, 0)),
        ],
        out_specs=pl.BlockSpec((L, Bb), lambda i: (0, i)),
        out_shape=jax.ShapeDtypeStruct((L, B), jnp.float32),
    )(z, mP, lvP, wPp, wrow)
    return out
